# baseline (speedup 1.0000x reference)
import functools

import jax
import jax.numpy as jnp
from jax import lax
from jax.experimental import pallas as pl
from jax.experimental.pallas import tpu as pltpu
from jax.experimental.pallas import tpu_sc as plsc

B = 16384
H = 50
D = 64
NC = 2
NS = 16
NW = NC * NS
BAGS_PER_W = B // NW
LANES = 16
NG = D // LANES


def _sc_bag_sum(x_r, table):
  mesh = plsc.VectorSubcoreMesh(core_axis_name="c", subcore_axis_name="s")

  @functools.partial(
      pl.kernel,
      out_type=jax.ShapeDtypeStruct((B, D), jnp.float32),
      mesh=mesh,
      compiler_params=pltpu.CompilerParams(use_tc_tiling_on_sc=False),
      scratch_types=[
          pltpu.VMEM((BAGS_PER_W, H), jnp.int32),
          pltpu.VMEM((H, D), jnp.float32),
          pltpu.VMEM((H, D), jnp.float32),
          pltpu.VMEM((BAGS_PER_W, D), jnp.float32),
          pltpu.SemaphoreType.DMA,
          pltpu.SemaphoreType.DMA,
      ],
  )
  def body(x_hbm, table_hbm, out_hbm, idx_v, rows0, rows1, out_v, sem0, sem1):
    wid = lax.axis_index("c") * NS + lax.axis_index("s")
    pltpu.sync_copy(x_hbm.at[wid], idx_v)

    def reduce_bag(buf, b):
      acc = [buf[0, pl.ds(g * LANES, LANES)] for g in range(NG)]
      for r in range(1, H):
        for g in range(NG):
          acc[g] = acc[g] + buf[r, pl.ds(g * LANES, LANES)]
      for g in range(NG):
        out_v[b, pl.ds(g * LANES, LANES)] = acc[g]

    def gather(b, buf, sem):
      return pltpu.async_copy(table_hbm.at[idx_v.at[b]], buf, sem)

    gather(0, rows0, sem0)

    def step(i, carry):
      b0 = i * 2
      gather(b0 + 1, rows1, sem1)
      pltpu.make_async_copy(table_hbm.at[idx_v.at[b0]], rows0, sem0).wait()
      reduce_bag(rows0, b0)
      gather(b0 + 2, rows0, sem0)
      pltpu.make_async_copy(table_hbm.at[idx_v.at[b0]], rows1, sem1).wait()
      reduce_bag(rows1, b0 + 1)
      return carry

    lax.fori_loop(0, BAGS_PER_W // 2 - 1, step, 0)
    b0 = BAGS_PER_W - 2
    gather(b0 + 1, rows1, sem1)
    pltpu.make_async_copy(table_hbm.at[idx_v.at[0]], rows0, sem0).wait()
    reduce_bag(rows0, b0)
    pltpu.make_async_copy(table_hbm.at[idx_v.at[0]], rows1, sem1).wait()
    reduce_bag(rows1, b0 + 1)

    pltpu.sync_copy(out_v, out_hbm.at[pl.ds(wid * BAGS_PER_W, BAGS_PER_W)])

  return body(x_r, table)


def _tc_mlp_softmax(s, w1, b1, w2, b2):
  TB = 2048

  def body(s_ref, w1_ref, b1_ref, w2_ref, b2_ref, o_ref):
    h = jnp.dot(s_ref[...], w1_ref[...], preferred_element_type=jnp.float32)
    h = h + b1_ref[...]
    h = jnp.dot(h, w2_ref[...], preferred_element_type=jnp.float32)
    h = h + b2_ref[...]
    m = jnp.max(h, axis=1, keepdims=True)
    e = jnp.exp(h - m)
    o_ref[...] = e / jnp.sum(e, axis=1, keepdims=True)

  return pl.pallas_call(
      body,
      out_shape=jax.ShapeDtypeStruct((B, D), jnp.float32),
      grid=(B // TB,),
      in_specs=[
          pl.BlockSpec((TB, D), lambda i: (i, 0)),
          pl.BlockSpec((D, D), lambda i: (0, 0)),
          pl.BlockSpec((1, D), lambda i: (0, 0)),
          pl.BlockSpec((D, D), lambda i: (0, 0)),
          pl.BlockSpec((1, D), lambda i: (0, 0)),
      ],
      out_specs=pl.BlockSpec((TB, D), lambda i: (i, 0)),
  )(s, w1, b1, w2, b2)


@jax.jit
def kernel(x, table, W1, b1, W2, b2):
  x_r = x.reshape(NW, BAGS_PER_W, H)
  sums = _sc_bag_sum(x_r, table)
  w1s = W1.T / float(H)
  return _tc_mlp_softmax(sums, w1s, b1.reshape(1, D), W2.T, b2.reshape(1, D))
